# baseline (speedup 1.0000x reference)
import functools

import jax
import jax.numpy as jnp
import numpy as np
from jax.experimental import pallas as pl
from jax.experimental.pallas import tpu as pltpu
from jax.sharding import Mesh, PartitionSpec as P
from jax.experimental.shard_map import shard_map


def _gru_cell_step(h, m, gi, whh, bhh, hid):
    h = h * m
    gh = jax.lax.dot_general(
        h.astype(jnp.bfloat16), whh,
        (((1,), (1,)), ((), ())),
        preferred_element_type=jnp.float32,
    ) + bhh
    r = jax.nn.sigmoid(gi[:, :hid] + gh[:, :hid])
    z = jax.nn.sigmoid(gi[:, hid:2 * hid] + gh[:, hid:2 * hid])
    n = jnp.tanh(gi[:, 2 * hid:] + r * gh[:, 2 * hid:])
    return (1.0 - z) * n + z * h


def _block_body(inp_ref, m_ref, wih_ref, whh_ref, bih_ref, bhh_ref, h_ref,
                yf_ref, yb_ref, hnew_ref, gi_s, *, steps, n_envs, hid):
    gi_s[...] = jax.lax.dot_general(
        inp_ref[...], wih_ref[...],
        (((1,), (1,)), ((), ())),
        preferred_element_type=jnp.float32,
    ) + bih_ref[...]

    whh = whh_ref[...]
    bhh = bhh_ref[...]

    h = h_ref[...]
    for b in range(steps):
        sl = slice(b * n_envs, (b + 1) * n_envs)
        h = _gru_cell_step(h, m_ref[b], gi_s[sl, :], whh, bhh, hid)
        yf_ref[sl, :] = h
        yb_ref[sl, :] = h.astype(jnp.bfloat16)
    hnew_ref[...] = h


def _block_call(inp, m, wih, whh, bih, bhh, h, *, steps, n_envs, hid):
    bn = steps * n_envs
    body = functools.partial(_block_body, steps=steps, n_envs=n_envs, hid=hid)
    return pl.pallas_call(
        body,
        out_shape=[
            jax.ShapeDtypeStruct((bn, hid), jnp.float32),
            jax.ShapeDtypeStruct((bn, hid), jnp.bfloat16),
            jax.ShapeDtypeStruct((n_envs, hid), jnp.float32),
        ],
        scratch_shapes=[
            pltpu.VMEM((bn, 3 * hid), jnp.float32),
        ],
    )(inp, m, wih, whh, bih, bhh, h)


def _mega_body(lidx_ref, x_ref, m0_ref, m1_ref,
               wih_ref, whh_ref, bih_ref, bhh_ref, h0_ref,
               y_ref, hout_ref,
               h_s, gi_s, ybuf_s, send_sem, recv_sem,
               *, steps, n_envs, hid, nblk, warm):
    k = pl.program_id(0)
    lidx = lidx_ref[0]
    peer = 1 - lidx

    def _copy(j):
        return pltpu.make_async_remote_copy(
            ybuf_s.at[j], ybuf_s.at[j], send_sem, recv_sem,
            device_id=peer, device_id_type=pltpu.DeviceIdType.LOGICAL)

    @pl.when(k == 0)
    def _():
        h_s[...] = h0_ref[...]
        barrier = pltpu.get_barrier_semaphore()
        pltpu.semaphore_signal(barrier, 1, device_id=peer,
                               device_id_type=pltpu.DeviceIdType.LOGICAL)

    @pl.when((lidx == 0) & (k < nblk))
    def _():
        gi_s[...] = jax.lax.dot_general(
            x_ref[...], wih_ref[...],
            (((1,), (1,)), ((), ())),
            preferred_element_type=jnp.float32,
        ) + bih_ref[...]
        whh = whh_ref[...]
        bhh = bhh_ref[...]
        h = h_s[...]
        for b in range(steps):
            sl = slice(b * n_envs, (b + 1) * n_envs)
            h = _gru_cell_step(h, m0_ref[b], gi_s[sl, :], whh, bhh, hid)
            ybuf_s[k, sl, :] = h.astype(jnp.bfloat16)
        h_s[...] = h
        hout_ref[...] = h

        @pl.when(k == warm)
        def _():
            barrier = pltpu.get_barrier_semaphore()
            pltpu.semaphore_wait(barrier, 1)
            for j in range(warm):
                _copy(j).start()

        @pl.when(k >= warm)
        def _():
            _copy(k).start()

    @pl.when((lidx == 0) & (k == nblk))
    def _():
        for j in range(nblk):
            _copy(j).wait_send()

    @pl.when((lidx == 1) & (k == 0))
    def _():
        barrier = pltpu.get_barrier_semaphore()
        pltpu.semaphore_wait(barrier, 1)

    @pl.when((lidx == 1) & (k > 0))
    def _():
        j = k - 1
        _copy(j).wait_recv()
        gi_s[...] = jax.lax.dot_general(
            ybuf_s[j], wih_ref[...],
            (((1,), (1,)), ((), ())),
            preferred_element_type=jnp.float32,
        ) + bih_ref[...]

        whh = whh_ref[...]
        bhh = bhh_ref[...]
        h = h_s[...]
        for b in range(steps):
            sl = slice(b * n_envs, (b + 1) * n_envs)
            h = _gru_cell_step(h, m1_ref[b], gi_s[sl, :], whh, bhh, hid)
            y_ref[sl, :] = h
        h_s[...] = h
        hout_ref[...] = h


def _dual_core(x, m3, W4, B4, h_init, *, t, n_envs, hid, block_t, devices):
    nblk = t // block_t
    bn = block_t * n_envs
    nstep = nblk + 1
    bf = jnp.bfloat16

    xb = x.astype(bf)
    lidx = jnp.arange(2, dtype=jnp.int32)
    mesh = Mesh(np.array(devices), ("c",))
    warm = min(12, nblk - 1)

    body = functools.partial(
        _mega_body, steps=block_t, n_envs=n_envs, hid=hid, nblk=nblk,
        warm=warm)

    def shard_body(lidx, xb, m3, W4, B4, h_init):
        full = lambda k: (0, 0)
        y, hout = pl.pallas_call(
            body,
            grid=(nstep,),
            in_specs=[
                pl.BlockSpec(memory_space=pltpu.SMEM),
                pl.BlockSpec((bn, hid), lambda k: (jnp.minimum(k, nblk - 1), 0)),
                pl.BlockSpec((block_t, n_envs, 1),
                             lambda k: (jnp.minimum(k, nblk - 1), 0, 0)),
                pl.BlockSpec((block_t, n_envs, 1),
                             lambda k: (jnp.maximum(k - 1, 0), 0, 0)),
                pl.BlockSpec((3 * hid, hid), full),
                pl.BlockSpec((3 * hid, hid), full),
                pl.BlockSpec((1, 3 * hid), full),
                pl.BlockSpec((1, 3 * hid), full),
                pl.BlockSpec((n_envs, hid), full),
            ],
            out_specs=[
                pl.BlockSpec((bn, hid), lambda k: (jnp.maximum(k - 1, 0), 0)),
                pl.BlockSpec((n_envs, hid), full),
            ],
            out_shape=[
                jax.ShapeDtypeStruct((t * n_envs, hid), jnp.float32),
                jax.ShapeDtypeStruct((n_envs, hid), jnp.float32),
            ],
            scratch_shapes=[
                pltpu.VMEM((n_envs, hid), jnp.float32),
                pltpu.VMEM((bn, 3 * hid), jnp.float32),
                pltpu.VMEM((nblk, bn, hid), jnp.bfloat16),
                pltpu.SemaphoreType.DMA,
                pltpu.SemaphoreType.DMA,
            ],
            compiler_params=pltpu.CompilerParams(
                dimension_semantics=("arbitrary",),
                collective_id=0,
            ),
        )(lidx, xb, m3, m3, W4[0, 0], W4[0, 1], B4[0, 0], B4[0, 1],
          h_init[0])
        return y[None], hout[None]

    y2, hout2 = shard_map(
        shard_body, mesh=mesh,
        in_specs=(P("c"), P(), P(), P("c"), P("c"), P("c")),
        out_specs=(P("c"), P("c")),
        check_rep=False,
    )(lidx, xb, m3, W4, B4, h_init)

    out = y2[1]
    hidden_out = jnp.stack([hout2[0], hout2[1]], axis=1)
    return out, hidden_out


def _single_core(x, m3, W4, B4, h_init, *, t, n_envs, hid, block_t):
    nblk = t // block_t
    bn = block_t * n_envs
    bf = jnp.bfloat16
    xb = x.astype(bf)

    def layer(inp, lidx, out_dtype):
        def body(x_ref, m_ref, wih_ref, whh_ref, bih_ref, bhh_ref, h0_ref,
                 y_ref, hout_ref, h_s, gi_s):
            i = pl.program_id(0)

            @pl.when(i == 0)
            def _():
                h_s[...] = h0_ref[...]

            gi_s[...] = jax.lax.dot_general(
                x_ref[...], wih_ref[...],
                (((1,), (1,)), ((), ())),
                preferred_element_type=jnp.float32,
            ) + bih_ref[...]
            whh = whh_ref[...]
            bhh = bhh_ref[...]
            h = h_s[...]
            for b in range(block_t):
                sl = slice(b * n_envs, (b + 1) * n_envs)
                h = _gru_cell_step(h, m_ref[b], gi_s[sl, :], whh, bhh, hid)
                y_ref[sl, :] = h.astype(out_dtype)
            h_s[...] = h
            hout_ref[...] = h

        full = lambda i: (0, 0)
        return pl.pallas_call(
            body,
            grid=(nblk,),
            in_specs=[
                pl.BlockSpec((bn, hid), lambda i: (i, 0)),
                pl.BlockSpec((block_t, n_envs, 1), lambda i: (i, 0, 0)),
                pl.BlockSpec((3 * hid, hid), full),
                pl.BlockSpec((3 * hid, hid), full),
                pl.BlockSpec((1, 3 * hid), full),
                pl.BlockSpec((1, 3 * hid), full),
                pl.BlockSpec((n_envs, hid), full),
            ],
            out_specs=[
                pl.BlockSpec((bn, hid), lambda i: (i, 0)),
                pl.BlockSpec((n_envs, hid), full),
            ],
            out_shape=[
                jax.ShapeDtypeStruct((t * n_envs, hid), out_dtype),
                jax.ShapeDtypeStruct((n_envs, hid), jnp.float32),
            ],
            scratch_shapes=[
                pltpu.VMEM((n_envs, hid), jnp.float32),
                pltpu.VMEM((bn, 3 * hid), jnp.float32),
            ],
            compiler_params=pltpu.CompilerParams(
                dimension_semantics=("arbitrary",),
            ),
        )(inp, m3, W4[lidx, 0], W4[lidx, 1],
          B4[lidx, 0], B4[lidx, 1], h_init[lidx])

    y0, h0f = layer(xb, 0, bf)
    y1, h1f = layer(y0, 1, jnp.float32)
    return y1, jnp.stack([h0f, h1f], axis=1)


def kernel(x, hidden_states, masks, W_ih_l0, W_hh_l0, b_ih_l0, b_hh_l0,
           W_ih_l1, W_hh_l1, b_ih_l1, b_hh_l1):
    n_envs, n_layers, hid = hidden_states.shape
    t = x.shape[0] // n_envs

    block_t = 16
    while t % block_t:
        block_t //= 2

    bf = jnp.bfloat16
    m3 = masks.reshape(t, n_envs, 1)
    W4 = jnp.stack([
        jnp.stack([W_ih_l0, W_hh_l0]),
        jnp.stack([W_ih_l1, W_hh_l1]),
    ]).astype(bf)
    B4 = jnp.stack([
        jnp.stack([b_ih_l0.reshape(1, -1), b_hh_l0.reshape(1, -1)]),
        jnp.stack([b_ih_l1.reshape(1, -1), b_hh_l1.reshape(1, -1)]),
    ])
    h_init = jnp.transpose(hidden_states, (1, 0, 2))

    devices = jax.devices()
    kw = dict(t=t, n_envs=n_envs, hid=hid, block_t=block_t)
    if len(devices) >= 2:
        return _dual_core(x, m3, W4, B4, h_init, devices=devices[:2], **kw)
    return _single_core(x, m3, W4, B4, h_init, **kw)


# speedup vs baseline: 1.4079x; 1.4079x over previous
import functools

import jax
import jax.numpy as jnp
import numpy as np
from jax.experimental import pallas as pl
from jax.experimental.pallas import tpu as pltpu
from jax.sharding import Mesh, PartitionSpec as P
from jax.experimental.shard_map import shard_map


def _gru_cell_step(h, m, gi, whh, bhh, hid):
    h = h * m
    gh = jax.lax.dot_general(
        h.astype(jnp.bfloat16), whh,
        (((1,), (0,)), ((), ())),
        preferred_element_type=jnp.float32,
    ) + bhh
    r = jax.nn.sigmoid(gi[:, :hid] + gh[:, :hid])
    z = jax.nn.sigmoid(gi[:, hid:2 * hid] + gh[:, hid:2 * hid])
    n = jnp.tanh(gi[:, 2 * hid:] + r * gh[:, 2 * hid:])
    return (1.0 - z) * n + z * h


def _block_body(inp_ref, m_ref, wih_ref, whh_ref, bih_ref, bhh_ref, h_ref,
                yf_ref, yb_ref, hnew_ref, gi_s, *, steps, n_envs, hid):
    gi_s[...] = jax.lax.dot_general(
        inp_ref[...], wih_ref[...],
        (((1,), (0,)), ((), ())),
        preferred_element_type=jnp.float32,
    ) + bih_ref[...]

    whh = whh_ref[...]
    bhh = bhh_ref[...]

    h = h_ref[...]
    for b in range(steps):
        sl = slice(b * n_envs, (b + 1) * n_envs)
        h = _gru_cell_step(h, m_ref[b], gi_s[sl, :], whh, bhh, hid)
        yf_ref[sl, :] = h
        yb_ref[sl, :] = h.astype(jnp.bfloat16)
    hnew_ref[...] = h


def _block_call(inp, m, wih, whh, bih, bhh, h, *, steps, n_envs, hid):
    bn = steps * n_envs
    body = functools.partial(_block_body, steps=steps, n_envs=n_envs, hid=hid)
    return pl.pallas_call(
        body,
        out_shape=[
            jax.ShapeDtypeStruct((bn, hid), jnp.float32),
            jax.ShapeDtypeStruct((bn, hid), jnp.bfloat16),
            jax.ShapeDtypeStruct((n_envs, hid), jnp.float32),
        ],
        scratch_shapes=[
            pltpu.VMEM((bn, 3 * hid), jnp.float32),
        ],
    )(inp, m, wih, whh, bih, bhh, h)


def _mega_body(lidx_ref, x_ref, m0_ref, m1_ref,
               wih_ref, whh_ref, bih_ref, bhh_ref, h0_ref,
               y_ref, hout_ref,
               h_s, gi_s, ybuf_s, send_sem, recv_sem,
               *, steps, n_envs, hid, nblk, warm):
    k = pl.program_id(0)
    lidx = lidx_ref[0]
    peer = 1 - lidx

    def _copy(j):
        return pltpu.make_async_remote_copy(
            ybuf_s.at[j], ybuf_s.at[j], send_sem, recv_sem,
            device_id=peer, device_id_type=pltpu.DeviceIdType.LOGICAL)

    @pl.when(k == 0)
    def _():
        h_s[...] = h0_ref[...]
        barrier = pltpu.get_barrier_semaphore()
        pltpu.semaphore_signal(barrier, 1, device_id=peer,
                               device_id_type=pltpu.DeviceIdType.LOGICAL)

    @pl.when((lidx == 0) & (k < nblk))
    def _():
        gi_s[...] = jax.lax.dot_general(
            x_ref[...], wih_ref[...],
            (((1,), (0,)), ((), ())),
            preferred_element_type=jnp.float32,
        ) + bih_ref[...]
        whh = whh_ref[...]
        bhh = bhh_ref[...]
        h = h_s[...]
        for b in range(steps):
            sl = slice(b * n_envs, (b + 1) * n_envs)
            h = _gru_cell_step(h, m0_ref[b], gi_s[sl, :], whh, bhh, hid)
            ybuf_s[k, sl, :] = h.astype(jnp.bfloat16)
        h_s[...] = h
        hout_ref[...] = h

        @pl.when(k == 0)
        def _():
            barrier = pltpu.get_barrier_semaphore()
            pltpu.semaphore_wait(barrier, 1)

        _copy(k).start()

    @pl.when((lidx == 0) & (k == nblk))
    def _():
        for j in range(nblk):
            _copy(j).wait_send()

    @pl.when((lidx == 1) & (k == 0))
    def _():
        barrier = pltpu.get_barrier_semaphore()
        pltpu.semaphore_wait(barrier, 1)

    @pl.when((lidx == 1) & (k > 0))
    def _():
        j = k - 1
        _copy(j).wait_recv()
        gi_s[...] = jax.lax.dot_general(
            ybuf_s[j], wih_ref[...],
            (((1,), (0,)), ((), ())),
            preferred_element_type=jnp.float32,
        ) + bih_ref[...]

        whh = whh_ref[...]
        bhh = bhh_ref[...]
        h = h_s[...]
        for b in range(steps):
            sl = slice(b * n_envs, (b + 1) * n_envs)
            h = _gru_cell_step(h, m1_ref[b], gi_s[sl, :], whh, bhh, hid)
            y_ref[sl, :] = h
        h_s[...] = h
        hout_ref[...] = h


def _dual_core(x, m3, W4, B4, h_init, *, t, n_envs, hid, block_t, devices):
    nblk = t // block_t
    bn = block_t * n_envs
    nstep = nblk + 1
    bf = jnp.bfloat16

    xb = x.astype(bf)
    lidx = jnp.arange(2, dtype=jnp.int32)
    mesh = Mesh(np.array(devices), ("c",))
    warm = min(12, nblk - 1)

    body = functools.partial(
        _mega_body, steps=block_t, n_envs=n_envs, hid=hid, nblk=nblk,
        warm=warm)

    def shard_body(lidx, xb, m3, W4, B4, h_init):
        full = lambda k: (0, 0)
        y, hout = pl.pallas_call(
            body,
            grid=(nstep,),
            in_specs=[
                pl.BlockSpec(memory_space=pltpu.SMEM),
                pl.BlockSpec((bn, hid), lambda k: (jnp.minimum(k, nblk - 1), 0)),
                pl.BlockSpec((block_t, n_envs, 1),
                             lambda k: (jnp.minimum(k, nblk - 1), 0, 0)),
                pl.BlockSpec((block_t, n_envs, 1),
                             lambda k: (jnp.maximum(k - 1, 0), 0, 0)),
                pl.BlockSpec((hid, 3 * hid), full),
                pl.BlockSpec((hid, 3 * hid), full),
                pl.BlockSpec((1, 3 * hid), full),
                pl.BlockSpec((1, 3 * hid), full),
                pl.BlockSpec((n_envs, hid), full),
            ],
            out_specs=[
                pl.BlockSpec((bn, hid), lambda k: (jnp.maximum(k - 1, 0), 0)),
                pl.BlockSpec((n_envs, hid), full),
            ],
            out_shape=[
                jax.ShapeDtypeStruct((t * n_envs, hid), jnp.float32),
                jax.ShapeDtypeStruct((n_envs, hid), jnp.float32),
            ],
            scratch_shapes=[
                pltpu.VMEM((n_envs, hid), jnp.float32),
                pltpu.VMEM((bn, 3 * hid), jnp.float32),
                pltpu.VMEM((nblk, bn, hid), jnp.bfloat16),
                pltpu.SemaphoreType.DMA,
                pltpu.SemaphoreType.DMA,
            ],
            compiler_params=pltpu.CompilerParams(
                dimension_semantics=("arbitrary",),
                collective_id=0,
            ),
        )(lidx, xb, m3, m3, W4[0, 0], W4[0, 1], B4[0, 0], B4[0, 1],
          h_init[0])
        return y[None], hout[None]

    y2, hout2 = shard_map(
        shard_body, mesh=mesh,
        in_specs=(P("c"), P(), P(), P("c"), P("c"), P("c")),
        out_specs=(P("c"), P("c")),
        check_rep=False,
    )(lidx, xb, m3, W4, B4, h_init)

    out = y2[1]
    hidden_out = jnp.stack([hout2[0], hout2[1]], axis=1)
    return out, hidden_out


def _single_core(x, m3, W4, B4, h_init, *, t, n_envs, hid, block_t):
    nblk = t // block_t
    bn = block_t * n_envs
    bf = jnp.bfloat16
    xb = x.astype(bf)

    def layer(inp, lidx, out_dtype):
        def body(x_ref, m_ref, wih_ref, whh_ref, bih_ref, bhh_ref, h0_ref,
                 y_ref, hout_ref, h_s, gi_s):
            i = pl.program_id(0)

            @pl.when(i == 0)
            def _():
                h_s[...] = h0_ref[...]

            gi_s[...] = jax.lax.dot_general(
                x_ref[...], wih_ref[...],
                (((1,), (0,)), ((), ())),
                preferred_element_type=jnp.float32,
            ) + bih_ref[...]
            whh = whh_ref[...]
            bhh = bhh_ref[...]
            h = h_s[...]
            for b in range(block_t):
                sl = slice(b * n_envs, (b + 1) * n_envs)
                h = _gru_cell_step(h, m_ref[b], gi_s[sl, :], whh, bhh, hid)
                y_ref[sl, :] = h.astype(out_dtype)
            h_s[...] = h
            hout_ref[...] = h

        full = lambda i: (0, 0)
        return pl.pallas_call(
            body,
            grid=(nblk,),
            in_specs=[
                pl.BlockSpec((bn, hid), lambda i: (i, 0)),
                pl.BlockSpec((block_t, n_envs, 1), lambda i: (i, 0, 0)),
                pl.BlockSpec((hid, 3 * hid), full),
                pl.BlockSpec((hid, 3 * hid), full),
                pl.BlockSpec((1, 3 * hid), full),
                pl.BlockSpec((1, 3 * hid), full),
                pl.BlockSpec((n_envs, hid), full),
            ],
            out_specs=[
                pl.BlockSpec((bn, hid), lambda i: (i, 0)),
                pl.BlockSpec((n_envs, hid), full),
            ],
            out_shape=[
                jax.ShapeDtypeStruct((t * n_envs, hid), out_dtype),
                jax.ShapeDtypeStruct((n_envs, hid), jnp.float32),
            ],
            scratch_shapes=[
                pltpu.VMEM((n_envs, hid), jnp.float32),
                pltpu.VMEM((bn, 3 * hid), jnp.float32),
            ],
            compiler_params=pltpu.CompilerParams(
                dimension_semantics=("arbitrary",),
            ),
        )(inp, m3, W4[lidx, 0], W4[lidx, 1],
          B4[lidx, 0], B4[lidx, 1], h_init[lidx])

    y0, h0f = layer(xb, 0, bf)
    y1, h1f = layer(y0, 1, jnp.float32)
    return y1, jnp.stack([h0f, h1f], axis=1)


def kernel(x, hidden_states, masks, W_ih_l0, W_hh_l0, b_ih_l0, b_hh_l0,
           W_ih_l1, W_hh_l1, b_ih_l1, b_hh_l1):
    n_envs, n_layers, hid = hidden_states.shape
    t = x.shape[0] // n_envs

    block_t = 16
    while t % block_t:
        block_t //= 2

    bf = jnp.bfloat16
    m3 = masks.reshape(t, n_envs, 1)
    W4 = jnp.stack([
        jnp.stack([W_ih_l0, W_hh_l0]),
        jnp.stack([W_ih_l1, W_hh_l1]),
    ]).astype(bf)
    W4 = jnp.swapaxes(W4, 2, 3)
    B4 = jnp.stack([
        jnp.stack([b_ih_l0.reshape(1, -1), b_hh_l0.reshape(1, -1)]),
        jnp.stack([b_ih_l1.reshape(1, -1), b_hh_l1.reshape(1, -1)]),
    ])
    h_init = jnp.transpose(hidden_states, (1, 0, 2))

    devices = jax.devices()
    kw = dict(t=t, n_envs=n_envs, hid=hid, block_t=block_t)
    if len(devices) >= 2:
        return _dual_core(x, m3, W4, B4, h_init, devices=devices[:2], **kw)
    return _single_core(x, m3, W4, B4, h_init, **kw)


# speedup vs baseline: 1.9055x; 1.3534x over previous
import functools

import jax
import jax.numpy as jnp
import numpy as np
from jax.experimental import pallas as pl
from jax.experimental.pallas import tpu as pltpu
from jax.sharding import Mesh, PartitionSpec as P
from jax.experimental.shard_map import shard_map


def _gru_cell_step(h, m, gi, whh, bhh, hid):
    h = h * m
    gh = jax.lax.dot_general(
        h.astype(jnp.bfloat16), whh,
        (((1,), (0,)), ((), ())),
        preferred_element_type=jnp.float32,
    ) + bhh
    r = jax.nn.sigmoid(gi[:, :hid] + gh[:, :hid])
    z = jax.nn.sigmoid(gi[:, hid:2 * hid] + gh[:, hid:2 * hid])
    n = jnp.tanh(gi[:, 2 * hid:] + r * gh[:, 2 * hid:])
    return (1.0 - z) * n + z * h


def _block_body(inp_ref, m_ref, wih_ref, whh_ref, bih_ref, bhh_ref, h_ref,
                yf_ref, yb_ref, hnew_ref, gi_s, *, steps, n_envs, hid):
    gi_s[...] = jax.lax.dot_general(
        inp_ref[...], wih_ref[...],
        (((1,), (0,)), ((), ())),
        preferred_element_type=jnp.float32,
    ) + bih_ref[...]

    whh = whh_ref[...]
    bhh = bhh_ref[...]

    h = h_ref[...]
    for b in range(steps):
        sl = slice(b * n_envs, (b + 1) * n_envs)
        h = _gru_cell_step(h, m_ref[b], gi_s[sl, :], whh, bhh, hid)
        yf_ref[sl, :] = h
        yb_ref[sl, :] = h.astype(jnp.bfloat16)
    hnew_ref[...] = h


def _block_call(inp, m, wih, whh, bih, bhh, h, *, steps, n_envs, hid):
    bn = steps * n_envs
    body = functools.partial(_block_body, steps=steps, n_envs=n_envs, hid=hid)
    return pl.pallas_call(
        body,
        out_shape=[
            jax.ShapeDtypeStruct((bn, hid), jnp.float32),
            jax.ShapeDtypeStruct((bn, hid), jnp.bfloat16),
            jax.ShapeDtypeStruct((n_envs, hid), jnp.float32),
        ],
        scratch_shapes=[
            pltpu.VMEM((bn, 3 * hid), jnp.float32),
        ],
    )(inp, m, wih, whh, bih, bhh, h)


def _mega_body(lidx_ref, x_ref, m0_ref, m1_ref,
               wih_ref, whh_ref, bih_ref, bhh_ref, h0_ref,
               y_ref, hout_ref,
               h_s, gi_s, ybuf_s, send_sem, recv_sem,
               *, steps, n_envs, hid, nblk, warm):
    k = pl.program_id(0)
    lidx = lidx_ref[0]
    peer = 1 - lidx

    def _copy(j):
        return pltpu.make_async_remote_copy(
            ybuf_s.at[j], ybuf_s.at[j], send_sem, recv_sem,
            device_id=peer, device_id_type=pltpu.DeviceIdType.LOGICAL)

    @pl.when(k == 0)
    def _():
        h_s[...] = h0_ref[...]
        barrier = pltpu.get_barrier_semaphore()
        pltpu.semaphore_signal(barrier, 1, device_id=peer,
                               device_id_type=pltpu.DeviceIdType.LOGICAL)

    @pl.when((lidx == 0) & (k < nblk))
    def _():
        gi_s[...] = jax.lax.dot_general(
            x_ref[...], wih_ref[...],
            (((1,), (0,)), ((), ())),
            preferred_element_type=jnp.float32,
        ) + bih_ref[...]
        whh = whh_ref[...]
        bhh = bhh_ref[...]
        h = h_s[...]
        for b in range(steps):
            sl = slice(b * n_envs, (b + 1) * n_envs)
            h = _gru_cell_step(h, m0_ref[b], gi_s[sl, :], whh, bhh, hid)
            ybuf_s[k, sl, :] = h.astype(jnp.bfloat16)
        h_s[...] = h
        hout_ref[...] = h

        @pl.when(k == 0)
        def _():
            barrier = pltpu.get_barrier_semaphore()
            pltpu.semaphore_wait(barrier, 1)

        _copy(k).start()

    @pl.when((lidx == 0) & (k == nblk))
    def _():
        for j in range(nblk):
            _copy(j).wait_send()

    @pl.when((lidx == 1) & (k == 0))
    def _():
        barrier = pltpu.get_barrier_semaphore()
        pltpu.semaphore_wait(barrier, 1)

    @pl.when((lidx == 1) & (k > 0))
    def _():
        j = k - 1
        _copy(j).wait_recv()
        gi_s[...] = jax.lax.dot_general(
            ybuf_s[j], wih_ref[...],
            (((1,), (0,)), ((), ())),
            preferred_element_type=jnp.float32,
        ) + bih_ref[...]

        whh = whh_ref[...]
        bhh = bhh_ref[...]
        h = h_s[...]
        for b in range(steps):
            sl = slice(b * n_envs, (b + 1) * n_envs)
            h = _gru_cell_step(h, m1_ref[b], gi_s[sl, :], whh, bhh, hid)
            y_ref[sl, :] = h
        h_s[...] = h
        hout_ref[...] = h


def _dual_core(x, m3, W4, B4, h_init, *, t, n_envs, hid, block_t, devices):
    nblk = t // block_t
    bn = block_t * n_envs
    nstep = nblk + 1
    bf = jnp.bfloat16

    xb = x.astype(bf)
    lidx = jnp.arange(2, dtype=jnp.int32)
    mesh = Mesh(np.array(devices), ("c",))
    warm = min(12, nblk - 1)

    body = functools.partial(
        _mega_body, steps=block_t, n_envs=n_envs, hid=hid, nblk=nblk,
        warm=warm)

    def shard_body(lidx, xb, m3, W4, B4, h_init):
        full = lambda k: (0, 0)
        y, hout = pl.pallas_call(
            body,
            grid=(nstep,),
            in_specs=[
                pl.BlockSpec(memory_space=pltpu.SMEM),
                pl.BlockSpec((bn, hid), lambda k: (jnp.minimum(k, nblk - 1), 0)),
                pl.BlockSpec((block_t, n_envs, 1),
                             lambda k: (jnp.minimum(k, nblk - 1), 0, 0)),
                pl.BlockSpec((block_t, n_envs, 1),
                             lambda k: (jnp.maximum(k - 1, 0), 0, 0)),
                pl.BlockSpec((hid, 3 * hid), full),
                pl.BlockSpec((hid, 3 * hid), full),
                pl.BlockSpec((1, 3 * hid), full),
                pl.BlockSpec((1, 3 * hid), full),
                pl.BlockSpec((n_envs, hid), full),
            ],
            out_specs=[
                pl.BlockSpec((bn, hid), lambda k: (jnp.maximum(k - 1, 0), 0)),
                pl.BlockSpec((n_envs, hid), full),
            ],
            out_shape=[
                jax.ShapeDtypeStruct((t * n_envs, hid), jnp.float32),
                jax.ShapeDtypeStruct((n_envs, hid), jnp.float32),
            ],
            scratch_shapes=[
                pltpu.VMEM((n_envs, hid), jnp.float32),
                pltpu.VMEM((bn, 3 * hid), jnp.float32),
                pltpu.VMEM((nblk, bn, hid), jnp.bfloat16),
                pltpu.SemaphoreType.DMA,
                pltpu.SemaphoreType.DMA,
            ],
            compiler_params=pltpu.CompilerParams(
                dimension_semantics=("arbitrary",),
                collective_id=0,
            ),
        )(lidx, xb, m3, m3, W4[0, 0], W4[0, 1], B4[0, 0], B4[0, 1],
          h_init[0])
        return y[None], hout[None]

    y2, hout2 = shard_map(
        shard_body, mesh=mesh,
        in_specs=(P("c"), P(), P(), P("c"), P("c"), P("c")),
        out_specs=(P("c"), P("c")),
        check_rep=False,
    )(lidx, xb, m3, W4, B4, h_init)

    out = y2[1]
    hidden_out = jnp.stack([hout2[0], hout2[1]], axis=1)
    return out, hidden_out


def _single_core(x, m3, W4, B4, h_init, *, t, n_envs, hid, block_t):
    nblk = t // block_t
    bn = block_t * n_envs
    bf = jnp.bfloat16
    xb = x.astype(bf)

    def layer(inp, lidx, out_dtype):
        def body(x_ref, m_ref, wih_ref, whh_ref, bih_ref, bhh_ref, h0_ref,
                 y_ref, hout_ref, h_s, gi_s):
            i = pl.program_id(0)

            @pl.when(i == 0)
            def _():
                h_s[...] = h0_ref[...]

            gi_s[...] = jax.lax.dot_general(
                x_ref[...], wih_ref[...],
                (((1,), (0,)), ((), ())),
                preferred_element_type=jnp.float32,
            ) + bih_ref[...]
            whh = whh_ref[...]
            bhh = bhh_ref[...]
            h = h_s[...]
            for b in range(block_t):
                sl = slice(b * n_envs, (b + 1) * n_envs)
                h = _gru_cell_step(h, m_ref[b], gi_s[sl, :], whh, bhh, hid)
                y_ref[sl, :] = h.astype(out_dtype)
            h_s[...] = h
            hout_ref[...] = h

        full = lambda i: (0, 0)
        return pl.pallas_call(
            body,
            grid=(nblk,),
            in_specs=[
                pl.BlockSpec((bn, hid), lambda i: (i, 0)),
                pl.BlockSpec((block_t, n_envs, 1), lambda i: (i, 0, 0)),
                pl.BlockSpec((hid, 3 * hid), full),
                pl.BlockSpec((hid, 3 * hid), full),
                pl.BlockSpec((1, 3 * hid), full),
                pl.BlockSpec((1, 3 * hid), full),
                pl.BlockSpec((n_envs, hid), full),
            ],
            out_specs=[
                pl.BlockSpec((bn, hid), lambda i: (i, 0)),
                pl.BlockSpec((n_envs, hid), full),
            ],
            out_shape=[
                jax.ShapeDtypeStruct((t * n_envs, hid), out_dtype),
                jax.ShapeDtypeStruct((n_envs, hid), jnp.float32),
            ],
            scratch_shapes=[
                pltpu.VMEM((n_envs, hid), jnp.float32),
                pltpu.VMEM((bn, 3 * hid), jnp.float32),
            ],
            compiler_params=pltpu.CompilerParams(
                dimension_semantics=("arbitrary",),
            ),
        )(inp, m3, W4[lidx, 0], W4[lidx, 1],
          B4[lidx, 0], B4[lidx, 1], h_init[lidx])

    y0, h0f = layer(xb, 0, bf)
    y1, h1f = layer(y0, 1, jnp.float32)
    return y1, jnp.stack([h0f, h1f], axis=1)


def kernel(x, hidden_states, masks, W_ih_l0, W_hh_l0, b_ih_l0, b_hh_l0,
           W_ih_l1, W_hh_l1, b_ih_l1, b_hh_l1):
    n_envs, n_layers, hid = hidden_states.shape
    t = x.shape[0] // n_envs

    block_t = 16
    while t % block_t:
        block_t //= 2

    bf = jnp.bfloat16
    m3 = masks.reshape(t, n_envs, 1)
    W4 = jnp.stack([
        jnp.stack([W_ih_l0, W_hh_l0]),
        jnp.stack([W_ih_l1, W_hh_l1]),
    ]).astype(bf)
    W4 = jnp.swapaxes(W4, 2, 3)
    B4 = jnp.stack([
        jnp.stack([b_ih_l0.reshape(1, -1), b_hh_l0.reshape(1, -1)]),
        jnp.stack([b_ih_l1.reshape(1, -1), b_hh_l1.reshape(1, -1)]),
    ])
    h_init = jnp.transpose(hidden_states, (1, 0, 2))

    devices = jax.devices()
    kw = dict(t=t, n_envs=n_envs, hid=hid, block_t=block_t)
    if False and len(devices) >= 2:
        return _dual_core(x, m3, W4, B4, h_init, devices=devices[:2], **kw)
    return _single_core(x, m3, W4, B4, h_init, **kw)
